# baseline (speedup 1.0000x reference)
import functools

import jax
import jax.numpy as jnp
from jax import lax
from jax.experimental import pallas as pl
from jax.experimental.pallas import tpu as pltpu
from jax.experimental.pallas import tpu_sc as plsc

V_DIM = 1000000
EMB_DIM = 64
BATCH = 16384
CTX = 20

NC = 2
NS = 16
NW = NC * NS

LANES = 16
PAIR_W = 2 * EMB_DIM
N_PAIRS = V_DIM // 2

VB = 128
SRC_STRIDE = VB + 1
N_FULL_BLOCKS = V_DIM // VB
MAIN_ITERS = (N_FULL_BLOCKS // NW // 2) * 2
N_MAIN_BLOCKS = MAIN_ITERS * NW
BLK_FLAT = (VB // 2) * PAIR_W

B_PER_W = BATCH // NW
OUT_PER_CHUNK = 4
ROWS_PER_CHUNK = OUT_PER_CHUNK * CTX
N_CHUNKS = B_PER_W // OUT_PER_CHUNK
NBUF = 4
COL_GROUPS = EMB_DIM // LANES
INV_CTX = 1.0 / CTX
OFF_W = 88


def _transpose_body(tT_hbm, out_hbm, s0_v, s1_v, d0_v, d1_v, st_v, dt_v,
                    sem_i0, sem_i1, sem_o0, sem_o1):
    s_v = (s0_v, s1_v)
    d_v = (d0_v, d1_v)
    sem_i = (sem_i0, sem_i1)
    sem_o = (sem_o0, sem_o1)
    wid = lax.axis_index("s") * NC + lax.axis_index("c")

    iota = lax.iota(jnp.int32, LANES)
    row_idx = [iota + ((g % 4) * LANES) for g in range(8)]

    def start_in(i, par):
        blk = wid + NW * i
        pltpu.async_copy(tT_hbm.at[:, pl.ds(blk * VB, VB)],
                         s_v[par].at[:, pl.ds(0, VB)], sem_i[par])

    def wait_in(par):
        pltpu.make_async_copy(tT_hbm.at[:, pl.ds(0, VB)],
                              s_v[par].at[:, pl.ds(0, VB)],
                              sem_i[par]).wait()

    def start_out(i, par):
        blk = wid + NW * i
        pltpu.async_copy(d_v[par], out_hbm.at[pl.ds(blk * BLK_FLAT,
                                                    BLK_FLAT)],
                         sem_o[par])

    def wait_out(par):
        pltpu.make_async_copy(d_v[par], out_hbm.at[pl.ds(0, BLK_FLAT)],
                              sem_o[par]).wait()

    def transpose_block(src, dst, n_pairs):
        @pl.loop(0, n_pairs, unroll=4)
        def _(p):
            base = p * PAIR_W
            for g in range(8):
                col = jnp.full((LANES,), 2 * p + (g // 4), jnp.int32)
                v = plsc.load_gather(src, [row_idx[g], col])
                dst[pl.ds(base + g * LANES, LANES)] = v

    start_in(0, 0)
    start_in(1, 1)

    @pl.loop(0, MAIN_ITERS, step=2)
    def _(i0):
        for par in range(2):
            i = i0 + par
            wait_in(par)

            @pl.when(i >= 2)
            def _():
                wait_out(par)

            transpose_block(s_v[par], d_v[par], VB // 2)
            start_out(i, par)

            @pl.when(i < MAIN_ITERS - 2)
            def _():
                start_in(i + 2, par)

    wait_out(0)
    wait_out(1)

    @pl.when(wid < 4)
    def _():
        blk = N_MAIN_BLOCKS + wid
        pltpu.sync_copy(tT_hbm.at[:, pl.ds(blk * VB, VB)],
                        s_v[0].at[:, pl.ds(0, VB)])
        transpose_block(s_v[0], d_v[0], VB // 2)
        pltpu.sync_copy(d_v[0], out_hbm.at[pl.ds(blk * BLK_FLAT,
                                                 BLK_FLAT)])

    @pl.when(wid == 4)
    def _():
        pltpu.sync_copy(tT_hbm.at[:, pl.ds(N_FULL_BLOCKS * VB,
                                           V_DIM - N_FULL_BLOCKS * VB)],
                        st_v)
        transpose_block(st_v, dt_v, (V_DIM - N_FULL_BLOCKS * VB) // 2)
        pltpu.sync_copy(
            dt_v,
            out_hbm.at[pl.ds(N_FULL_BLOCKS * BLK_FLAT,
                             (V_DIM - N_FULL_BLOCKS * VB) // 2 * PAIR_W)])


def _cbow_body(pair_hbm, off_hbm, table_hbm, out_hbm,
               pair_v, off_v, bufs, out_v, sem0, sem1, sem2, sem3):
    sems = (sem0, sem1, sem2, sem3)
    wid = lax.axis_index("s") * NC + lax.axis_index("c")

    pltpu.sync_copy(pair_hbm.at[wid], pair_v)
    pltpu.sync_copy(off_hbm.at[wid], off_v)

    def start_gather(c, b):
        pltpu.async_copy(table_hbm.at[pair_v.at[c]], bufs.at[b], sems[b])

    def wait_gather(b):
        pltpu.make_async_copy(
            table_hbm.at[pair_v.at[0]], bufs.at[b], sems[b]).wait()

    iota = lax.iota(jnp.int32, LANES)
    col_base = [iota + (g * LANES) for g in range(COL_GROUPS)]

    def reduce_chunk(c, b):
        buf = bufs.at[b]
        off_row = off_v.at[c]
        for j in range(OUT_PER_CHUNK):
            out_base = (c * OUT_PER_CHUNK + j) * EMB_DIM
            acc = [None] * COL_GROUPS
            for k in range(CTX):
                r = j * CTX + k
                rsplat = jnp.full((LANES,), r, jnp.int32)
                half = plsc.load_gather(off_row, [jnp.full((LANES,), r + 1,
                                                          jnp.int32)])
                for g in range(COL_GROUPS):
                    v = plsc.load_gather(buf, [rsplat, half + col_base[g]])
                    acc[g] = v if k == 0 else acc[g] + v
            for g in range(COL_GROUPS):
                out_v[pl.ds(out_base + g * LANES, LANES)] = acc[g] * INV_CTX

    for b in range(NBUF):
        start_gather(b, b)

    @pl.loop(0, N_CHUNKS, step=NBUF)
    def _(cc):
        for b in range(NBUF):
            c = cc + b
            wait_gather(b)
            reduce_chunk(c, b)

            @pl.when(c < N_CHUNKS - NBUF)
            def _():
                start_gather(c + NBUF, b)

    pltpu.sync_copy(out_v, out_hbm.at[pl.ds(wid * B_PER_W * EMB_DIM,
                                            B_PER_W * EMB_DIM)])


@jax.jit
def _cbow_sc(x, emb_table):
    xi = x.astype(jnp.int32)
    pair_grouped = (xi >> 1).reshape(NW, N_CHUNKS, ROWS_PER_CHUNK)
    off_grouped = jnp.pad(
        ((xi & 1) * EMB_DIM).reshape(NW, N_CHUNKS, ROWS_PER_CHUNK),
        ((0, 0), (0, 0), (1, OFF_W - ROWS_PER_CHUNK - 1)))

    mesh = plsc.VectorSubcoreMesh(core_axis_name="c", subcore_axis_name="s")

    transpose_run = pl.kernel(
        _transpose_body,
        out_type=jax.ShapeDtypeStruct((N_PAIRS * PAIR_W,), jnp.float32),
        mesh=mesh,
        scratch_types=[
            pltpu.VMEM((EMB_DIM, SRC_STRIDE), jnp.float32),
            pltpu.VMEM((EMB_DIM, SRC_STRIDE), jnp.float32),
            pltpu.VMEM((BLK_FLAT,), jnp.float32),
            pltpu.VMEM((BLK_FLAT,), jnp.float32),
            pltpu.VMEM((EMB_DIM, V_DIM - N_FULL_BLOCKS * VB), jnp.float32),
            pltpu.VMEM(((V_DIM - N_FULL_BLOCKS * VB) // 2 * PAIR_W,),
                       jnp.float32),
            pltpu.SemaphoreType.DMA,
            pltpu.SemaphoreType.DMA,
            pltpu.SemaphoreType.DMA,
            pltpu.SemaphoreType.DMA,
        ],
        compiler_params=pltpu.CompilerParams(needs_layout_passes=False),
    )
    table_pairs = transpose_run(emb_table.T).reshape(N_PAIRS, PAIR_W)

    gather_run = pl.kernel(
        _cbow_body,
        out_type=jax.ShapeDtypeStruct((BATCH * EMB_DIM,), jnp.float32),
        mesh=mesh,
        scratch_types=[
            pltpu.VMEM((N_CHUNKS, ROWS_PER_CHUNK), jnp.int32),
            pltpu.VMEM((N_CHUNKS, OFF_W), jnp.int32),
            pltpu.VMEM((NBUF, ROWS_PER_CHUNK, PAIR_W), jnp.float32),
            pltpu.VMEM((B_PER_W * EMB_DIM,), jnp.float32),
            pltpu.SemaphoreType.DMA,
            pltpu.SemaphoreType.DMA,
            pltpu.SemaphoreType.DMA,
            pltpu.SemaphoreType.DMA,
        ],
        compiler_params=pltpu.CompilerParams(needs_layout_passes=False),
    )
    out = gather_run(pair_grouped, off_grouped, table_pairs)
    return out.reshape(BATCH, EMB_DIM)


def kernel(x, emb_table):
    return _cbow_sc(x, emb_table)


# speedup vs baseline: 1.3106x; 1.3106x over previous
import functools

import jax
import jax.numpy as jnp
from jax import lax
from jax.experimental import pallas as pl
from jax.experimental.pallas import tpu as pltpu
from jax.experimental.pallas import tpu_sc as plsc

V_DIM = 1000000
EMB_DIM = 64
BATCH = 16384
CTX = 20

NC = 2
NS = 16
NW = NC * NS

LANES = 16
PAIR_W = 2 * EMB_DIM
N_PAIRS = V_DIM // 2

VB = 128
SRC_STRIDE = VB + 1
N_FULL_BLOCKS = V_DIM // VB
MAIN_ITERS = (N_FULL_BLOCKS // NW // 2) * 2
N_MAIN_BLOCKS = MAIN_ITERS * NW
BLK_FLAT = (VB // 2) * PAIR_W

B_PER_W = BATCH // NW
OUT_PER_CHUNK = 4
ROWS_PER_CHUNK = OUT_PER_CHUNK * CTX
N_CHUNKS = B_PER_W // OUT_PER_CHUNK
NBUF = 4
COL_GROUPS = EMB_DIM // LANES
INV_CTX = 1.0 / CTX
OFF_W = 88


def _transpose_body(tT_hbm, out_hbm, s0_v, s1_v, d0_v, d1_v, st_v, dt_v,
                    sem_i0, sem_i1, sem_o0, sem_o1):
    s_v = (s0_v, s1_v)
    d_v = (d0_v, d1_v)
    sem_i = (sem_i0, sem_i1)
    sem_o = (sem_o0, sem_o1)
    wid = lax.axis_index("s") * NC + lax.axis_index("c")

    iota = lax.iota(jnp.int32, LANES)
    row_idx = [iota + ((g % 4) * LANES) for g in range(8)]

    def start_in(i, par):
        blk = wid + NW * i
        pltpu.async_copy(tT_hbm.at[:, pl.ds(blk * VB, VB)],
                         s_v[par].at[:, pl.ds(0, VB)], sem_i[par])

    def wait_in(par):
        pltpu.make_async_copy(tT_hbm.at[:, pl.ds(0, VB)],
                              s_v[par].at[:, pl.ds(0, VB)],
                              sem_i[par]).wait()

    def start_out(i, par):
        blk = wid + NW * i
        pltpu.async_copy(d_v[par], out_hbm.at[pl.ds(blk * BLK_FLAT,
                                                    BLK_FLAT)],
                         sem_o[par])

    def wait_out(par):
        pltpu.make_async_copy(d_v[par], out_hbm.at[pl.ds(0, BLK_FLAT)],
                              sem_o[par]).wait()

    half_off = (jnp.zeros((LANES,), jnp.int32),
                jnp.full((LANES,), 1, jnp.int32))

    def transpose_block(src, dst, n_pairs):
        @pl.loop(0, n_pairs, step=4)
        def _(p0):
            vals = []
            for dp in range(4):
                p = p0 + dp
                psplat = jnp.full((LANES,), 2 * p, jnp.int32)
                cols = (psplat + half_off[0], psplat + half_off[1])
                for g in range(8):
                    vals.append(
                        (p * PAIR_W + g * LANES,
                         plsc.load_gather(src, [row_idx[g], cols[g // 4]])))
            for off, v in vals:
                dst[pl.ds(off, LANES)] = v

    start_in(0, 0)
    start_in(1, 1)

    @pl.loop(0, MAIN_ITERS, step=2)
    def _(i0):
        for par in range(2):
            i = i0 + par
            wait_in(par)

            @pl.when(i >= 2)
            def _():
                wait_out(par)

            transpose_block(s_v[par], d_v[par], VB // 2)
            start_out(i, par)

            @pl.when(i < MAIN_ITERS - 2)
            def _():
                start_in(i + 2, par)

    wait_out(0)
    wait_out(1)

    @pl.when(wid < 4)
    def _():
        blk = N_MAIN_BLOCKS + wid
        pltpu.sync_copy(tT_hbm.at[:, pl.ds(blk * VB, VB)],
                        s_v[0].at[:, pl.ds(0, VB)])
        transpose_block(s_v[0], d_v[0], VB // 2)
        pltpu.sync_copy(d_v[0], out_hbm.at[pl.ds(blk * BLK_FLAT,
                                                 BLK_FLAT)])

    @pl.when(wid == 4)
    def _():
        pltpu.sync_copy(tT_hbm.at[:, pl.ds(N_FULL_BLOCKS * VB,
                                           V_DIM - N_FULL_BLOCKS * VB)],
                        st_v)
        transpose_block(st_v, dt_v, (V_DIM - N_FULL_BLOCKS * VB) // 2)
        pltpu.sync_copy(
            dt_v,
            out_hbm.at[pl.ds(N_FULL_BLOCKS * BLK_FLAT,
                             (V_DIM - N_FULL_BLOCKS * VB) // 2 * PAIR_W)])


def _cbow_body(pair_hbm, off_hbm, table_hbm, out_hbm,
               pair_v, off_v, bufs, out_v, sem0, sem1, sem2, sem3):
    sems = (sem0, sem1, sem2, sem3)
    wid = lax.axis_index("s") * NC + lax.axis_index("c")

    pltpu.sync_copy(pair_hbm.at[wid], pair_v)
    pltpu.sync_copy(off_hbm.at[wid], off_v)

    def start_gather(c, b):
        pltpu.async_copy(table_hbm.at[pair_v.at[c]], bufs.at[b], sems[b])

    def wait_gather(b):
        pltpu.make_async_copy(
            table_hbm.at[pair_v.at[0]], bufs.at[b], sems[b]).wait()

    iota = lax.iota(jnp.int32, LANES)
    col_base = [iota + (g * LANES) for g in range(COL_GROUPS)]

    def reduce_chunk(c, b):
        buf = bufs.at[b]
        off_row = off_v.at[c]
        for j in range(OUT_PER_CHUNK):
            out_base = (c * OUT_PER_CHUNK + j) * EMB_DIM
            acc = [None] * COL_GROUPS
            for k in range(CTX):
                r = j * CTX + k
                rsplat = jnp.full((LANES,), r, jnp.int32)
                half = plsc.load_gather(off_row, [jnp.full((LANES,), r + 1,
                                                          jnp.int32)])
                for g in range(COL_GROUPS):
                    v = plsc.load_gather(buf, [rsplat, half + col_base[g]])
                    acc[g] = v if k == 0 else acc[g] + v
            for g in range(COL_GROUPS):
                out_v[pl.ds(out_base + g * LANES, LANES)] = acc[g] * INV_CTX

    for b in range(NBUF):
        start_gather(b, b)

    @pl.loop(0, N_CHUNKS, step=NBUF)
    def _(cc):
        for b in range(NBUF):
            c = cc + b
            wait_gather(b)
            reduce_chunk(c, b)

            @pl.when(c < N_CHUNKS - NBUF)
            def _():
                start_gather(c + NBUF, b)

    pltpu.sync_copy(out_v, out_hbm.at[pl.ds(wid * B_PER_W * EMB_DIM,
                                            B_PER_W * EMB_DIM)])


@jax.jit
def _cbow_sc(x, emb_table):
    xi = x.astype(jnp.int32)
    pair_grouped = (xi >> 1).reshape(NW, N_CHUNKS, ROWS_PER_CHUNK)
    off_grouped = jnp.pad(
        ((xi & 1) * EMB_DIM).reshape(NW, N_CHUNKS, ROWS_PER_CHUNK),
        ((0, 0), (0, 0), (1, OFF_W - ROWS_PER_CHUNK - 1)))

    mesh = plsc.VectorSubcoreMesh(core_axis_name="c", subcore_axis_name="s")

    transpose_run = pl.kernel(
        _transpose_body,
        out_type=jax.ShapeDtypeStruct((N_PAIRS * PAIR_W,), jnp.float32),
        mesh=mesh,
        scratch_types=[
            pltpu.VMEM((EMB_DIM, SRC_STRIDE), jnp.float32),
            pltpu.VMEM((EMB_DIM, SRC_STRIDE), jnp.float32),
            pltpu.VMEM((BLK_FLAT,), jnp.float32),
            pltpu.VMEM((BLK_FLAT,), jnp.float32),
            pltpu.VMEM((EMB_DIM, V_DIM - N_FULL_BLOCKS * VB), jnp.float32),
            pltpu.VMEM(((V_DIM - N_FULL_BLOCKS * VB) // 2 * PAIR_W,),
                       jnp.float32),
            pltpu.SemaphoreType.DMA,
            pltpu.SemaphoreType.DMA,
            pltpu.SemaphoreType.DMA,
            pltpu.SemaphoreType.DMA,
        ],
        compiler_params=pltpu.CompilerParams(needs_layout_passes=False),
    )
    table_pairs = transpose_run(emb_table.T).reshape(N_PAIRS, PAIR_W)

    gather_run = pl.kernel(
        _cbow_body,
        out_type=jax.ShapeDtypeStruct((BATCH * EMB_DIM,), jnp.float32),
        mesh=mesh,
        scratch_types=[
            pltpu.VMEM((N_CHUNKS, ROWS_PER_CHUNK), jnp.int32),
            pltpu.VMEM((N_CHUNKS, OFF_W), jnp.int32),
            pltpu.VMEM((NBUF, ROWS_PER_CHUNK, PAIR_W), jnp.float32),
            pltpu.VMEM((B_PER_W * EMB_DIM,), jnp.float32),
            pltpu.SemaphoreType.DMA,
            pltpu.SemaphoreType.DMA,
            pltpu.SemaphoreType.DMA,
            pltpu.SemaphoreType.DMA,
        ],
        compiler_params=pltpu.CompilerParams(needs_layout_passes=False),
    )
    out = gather_run(pair_grouped, off_grouped, table_pairs)
    return out.reshape(BATCH, EMB_DIM)


def kernel(x, emb_table):
    return _cbow_sc(x, emb_table)


# speedup vs baseline: 1.3146x; 1.0030x over previous
import functools

import jax
import jax.numpy as jnp
from jax import lax
from jax.experimental import pallas as pl
from jax.experimental.pallas import tpu as pltpu
from jax.experimental.pallas import tpu_sc as plsc

V_DIM = 1000000
EMB_DIM = 64
BATCH = 16384
CTX = 20

NC = 2
NS = 16
NW = NC * NS

LANES = 16
PAIR_W = 2 * EMB_DIM
N_PAIRS = V_DIM // 2

VB = 128
SRC_STRIDE = VB + 8
N_FULL_BLOCKS = V_DIM // VB
MAIN_ITERS = (N_FULL_BLOCKS // NW // 2) * 2
N_MAIN_BLOCKS = MAIN_ITERS * NW
BLK_FLAT = (VB // 2) * PAIR_W

B_PER_W = BATCH // NW
OUT_PER_CHUNK = 4
ROWS_PER_CHUNK = OUT_PER_CHUNK * CTX
N_CHUNKS = B_PER_W // OUT_PER_CHUNK
NBUF = 4
COL_GROUPS = EMB_DIM // LANES
INV_CTX = 1.0 / CTX
OFF_W = 88


def _transpose_body(tT_hbm, out_hbm, s0_v, s1_v, d0_v, d1_v, st_v, dt_v,
                    sem_i0, sem_i1, sem_o0, sem_o1):
    s_v = (s0_v, s1_v)
    d_v = (d0_v, d1_v)
    sem_i = (sem_i0, sem_i1)
    sem_o = (sem_o0, sem_o1)
    wid = lax.axis_index("s") * NC + lax.axis_index("c")

    iota = lax.iota(jnp.int32, LANES)
    row_idx = [iota + ((g % 4) * LANES) for g in range(8)]

    def start_in(i, par):
        blk = wid + NW * i
        pltpu.async_copy(tT_hbm.at[:, pl.ds(blk * VB, VB)],
                         s_v[par].at[:, pl.ds(0, VB)], sem_i[par])

    def wait_in(par):
        pltpu.make_async_copy(tT_hbm.at[:, pl.ds(0, VB)],
                              s_v[par].at[:, pl.ds(0, VB)],
                              sem_i[par]).wait()

    def start_out(i, par):
        blk = wid + NW * i
        pltpu.async_copy(d_v[par], out_hbm.at[pl.ds(blk * BLK_FLAT,
                                                    BLK_FLAT)],
                         sem_o[par])

    def wait_out(par):
        pltpu.make_async_copy(d_v[par], out_hbm.at[pl.ds(0, BLK_FLAT)],
                              sem_o[par]).wait()

    half_off = (jnp.zeros((LANES,), jnp.int32),
                jnp.full((LANES,), 1, jnp.int32))

    def transpose_block(src, dst, n_pairs):
        @pl.loop(0, n_pairs, step=4)
        def _(p0):
            vals = []
            for dp in range(4):
                p = p0 + dp
                psplat = jnp.full((LANES,), 2 * p, jnp.int32)
                cols = (psplat + half_off[0], psplat + half_off[1])
                for g in range(8):
                    vals.append(
                        (p * PAIR_W + g * LANES,
                         plsc.load_gather(src, [row_idx[g], cols[g // 4]])))
            for off, v in vals:
                dst[pl.ds(off, LANES)] = v

    start_in(0, 0)
    start_in(1, 1)

    @pl.loop(0, MAIN_ITERS, step=2)
    def _(i0):
        for par in range(2):
            i = i0 + par
            wait_in(par)

            @pl.when(i >= 2)
            def _():
                wait_out(par)

            transpose_block(s_v[par], d_v[par], VB // 2)
            start_out(i, par)

            @pl.when(i < MAIN_ITERS - 2)
            def _():
                start_in(i + 2, par)

    wait_out(0)
    wait_out(1)

    @pl.when(wid < 4)
    def _():
        blk = N_MAIN_BLOCKS + wid
        pltpu.sync_copy(tT_hbm.at[:, pl.ds(blk * VB, VB)],
                        s_v[0].at[:, pl.ds(0, VB)])
        transpose_block(s_v[0], d_v[0], VB // 2)
        pltpu.sync_copy(d_v[0], out_hbm.at[pl.ds(blk * BLK_FLAT,
                                                 BLK_FLAT)])

    @pl.when(wid == 4)
    def _():
        pltpu.sync_copy(tT_hbm.at[:, pl.ds(N_FULL_BLOCKS * VB,
                                           V_DIM - N_FULL_BLOCKS * VB)],
                        st_v)
        transpose_block(st_v, dt_v, (V_DIM - N_FULL_BLOCKS * VB) // 2)
        pltpu.sync_copy(
            dt_v,
            out_hbm.at[pl.ds(N_FULL_BLOCKS * BLK_FLAT,
                             (V_DIM - N_FULL_BLOCKS * VB) // 2 * PAIR_W)])


def _cbow_body(pair_hbm, off_hbm, table_hbm, out_hbm,
               pair_v, off_v, bufs, out_v, sem0, sem1, sem2, sem3):
    sems = (sem0, sem1, sem2, sem3)
    wid = lax.axis_index("s") * NC + lax.axis_index("c")

    pltpu.sync_copy(pair_hbm.at[wid], pair_v)
    pltpu.sync_copy(off_hbm.at[wid], off_v)

    def start_gather(c, b):
        pltpu.async_copy(table_hbm.at[pair_v.at[c]], bufs.at[b], sems[b])

    def wait_gather(b):
        pltpu.make_async_copy(
            table_hbm.at[pair_v.at[0]], bufs.at[b], sems[b]).wait()

    iota = lax.iota(jnp.int32, LANES)
    col_base = [iota + (g * LANES) for g in range(COL_GROUPS)]

    def reduce_chunk(c, b):
        buf = bufs.at[b]
        off_row = off_v.at[c]
        for j in range(OUT_PER_CHUNK):
            out_base = (c * OUT_PER_CHUNK + j) * EMB_DIM
            acc = [None] * COL_GROUPS
            for k in range(CTX):
                r = j * CTX + k
                rsplat = jnp.full((LANES,), r, jnp.int32)
                half = plsc.load_gather(off_row, [jnp.full((LANES,), r + 1,
                                                          jnp.int32)])
                for g in range(COL_GROUPS):
                    v = plsc.load_gather(buf, [rsplat, half + col_base[g]])
                    acc[g] = v if k == 0 else acc[g] + v
            for g in range(COL_GROUPS):
                out_v[pl.ds(out_base + g * LANES, LANES)] = acc[g] * INV_CTX

    for b in range(NBUF):
        start_gather(b, b)

    @pl.loop(0, N_CHUNKS, step=NBUF)
    def _(cc):
        for b in range(NBUF):
            c = cc + b
            wait_gather(b)
            reduce_chunk(c, b)

            @pl.when(c < N_CHUNKS - NBUF)
            def _():
                start_gather(c + NBUF, b)

    pltpu.sync_copy(out_v, out_hbm.at[pl.ds(wid * B_PER_W * EMB_DIM,
                                            B_PER_W * EMB_DIM)])


@jax.jit
def _cbow_sc(x, emb_table):
    xi = x.astype(jnp.int32)
    pair_grouped = (xi >> 1).reshape(NW, N_CHUNKS, ROWS_PER_CHUNK)
    off_grouped = jnp.pad(
        ((xi & 1) * EMB_DIM).reshape(NW, N_CHUNKS, ROWS_PER_CHUNK),
        ((0, 0), (0, 0), (1, OFF_W - ROWS_PER_CHUNK - 1)))

    mesh = plsc.VectorSubcoreMesh(core_axis_name="c", subcore_axis_name="s")

    transpose_run = pl.kernel(
        _transpose_body,
        out_type=jax.ShapeDtypeStruct((N_PAIRS * PAIR_W,), jnp.float32),
        mesh=mesh,
        scratch_types=[
            pltpu.VMEM((EMB_DIM, SRC_STRIDE), jnp.float32),
            pltpu.VMEM((EMB_DIM, SRC_STRIDE), jnp.float32),
            pltpu.VMEM((BLK_FLAT,), jnp.float32),
            pltpu.VMEM((BLK_FLAT,), jnp.float32),
            pltpu.VMEM((EMB_DIM, V_DIM - N_FULL_BLOCKS * VB), jnp.float32),
            pltpu.VMEM(((V_DIM - N_FULL_BLOCKS * VB) // 2 * PAIR_W,),
                       jnp.float32),
            pltpu.SemaphoreType.DMA,
            pltpu.SemaphoreType.DMA,
            pltpu.SemaphoreType.DMA,
            pltpu.SemaphoreType.DMA,
        ],
        compiler_params=pltpu.CompilerParams(needs_layout_passes=False),
    )
    table_pairs = transpose_run(emb_table.T).reshape(N_PAIRS, PAIR_W)

    gather_run = pl.kernel(
        _cbow_body,
        out_type=jax.ShapeDtypeStruct((BATCH * EMB_DIM,), jnp.float32),
        mesh=mesh,
        scratch_types=[
            pltpu.VMEM((N_CHUNKS, ROWS_PER_CHUNK), jnp.int32),
            pltpu.VMEM((N_CHUNKS, OFF_W), jnp.int32),
            pltpu.VMEM((NBUF, ROWS_PER_CHUNK, PAIR_W), jnp.float32),
            pltpu.VMEM((B_PER_W * EMB_DIM,), jnp.float32),
            pltpu.SemaphoreType.DMA,
            pltpu.SemaphoreType.DMA,
            pltpu.SemaphoreType.DMA,
            pltpu.SemaphoreType.DMA,
        ],
        compiler_params=pltpu.CompilerParams(needs_layout_passes=False),
    )
    out = gather_run(pair_grouped, off_grouped, table_pairs)
    return out.reshape(BATCH, EMB_DIM)


def kernel(x, emb_table):
    return _cbow_sc(x, emb_table)


# speedup vs baseline: 3.4957x; 2.6592x over previous
import functools

import jax
import jax.numpy as jnp
from jax import lax
from jax.experimental import pallas as pl
from jax.experimental.pallas import tpu as pltpu
from jax.experimental.pallas import tpu_sc as plsc

V_DIM = 1000000
EMB_DIM = 64
BATCH = 16384
CTX = 20

NC = 2
NS = 16
NW = NC * NS

LANES = 16
PAIR_W = 2 * EMB_DIM
N_PAIRS = V_DIM // 2

VB = 128
SRC_STRIDE = VB
N_FULL_BLOCKS = V_DIM // VB
MAIN_ITERS = (N_FULL_BLOCKS // NW // 2) * 2
N_MAIN_BLOCKS = MAIN_ITERS * NW
BLK_FLAT = (VB // 2) * PAIR_W

B_PER_W = BATCH // NW
OUT_PER_CHUNK = 4
ROWS_PER_CHUNK = OUT_PER_CHUNK * CTX
N_CHUNKS = B_PER_W // OUT_PER_CHUNK
NBUF = 4
COL_GROUPS = EMB_DIM // LANES
INV_CTX = 1.0 / CTX
OFF_W = 88


def _transpose_body(tT_hbm, out_hbm, s0_v, s1_v, d0_v, d1_v, st_v, dt_v,
                    sem_i0, sem_i1, sem_o0, sem_o1):
    s_v = (s0_v, s1_v)
    d_v = (d0_v, d1_v)
    sem_i = (sem_i0, sem_i1)
    sem_o = (sem_o0, sem_o1)
    wid = lax.axis_index("s") * NC + lax.axis_index("c")

    iota = lax.iota(jnp.int32, LANES)

    def start_in(i, par):
        blk = wid + NW * i
        pltpu.async_copy(tT_hbm.at[:, pl.ds(blk * VB, VB)], s_v[par],
                         sem_i[par])

    def wait_in(par):
        pltpu.make_async_copy(tT_hbm.at[:, pl.ds(0, VB)], s_v[par],
                              sem_i[par]).wait()

    def start_out(i, par):
        blk = wid + NW * i
        pltpu.async_copy(d_v[par], out_hbm.at[pl.ds(blk * BLK_FLAT,
                                                    BLK_FLAT)],
                         sem_o[par])

    def wait_out(par):
        pltpu.make_async_copy(d_v[par], out_hbm.at[pl.ds(0, BLK_FLAT)],
                              sem_o[par]).wait()

    dn = lax.GatherDimensionNumbers(offset_dims=(), collapsed_slice_dims=(0,),
                                    start_index_map=(0,))

    def lane_xor(v, s):
        return lax.gather(v, (iota ^ s).reshape(LANES, 1), dn, (1,),
                          mode=lax.GatherScatterMode.PROMISE_IN_BOUNDS)

    def transpose16(a):
        cur = a
        for s in (1, 2, 4, 8):
            nxt = [None] * LANES
            for i in range(LANES):
                keep = (iota & s) == (i & s)
                nxt[i] = jnp.where(keep, cur[i], lane_xor(cur[i ^ s], s))
            cur = nxt
        return cur

    def transpose_block(src, dst, n_vcol):
        @pl.loop(0, n_vcol)
        def _(vc):
            for frow in range(EMB_DIM // LANES):
                a = [src[LANES * frow + r, pl.ds(vc * LANES, LANES)]
                     for r in range(LANES)]
                b = transpose16(a)
                for c in range(LANES):
                    dst[pl.ds((vc * LANES + c) * EMB_DIM + LANES * frow,
                              LANES)] = b[c]

    start_in(0, 0)
    start_in(1, 1)

    @pl.loop(0, MAIN_ITERS, step=2)
    def _(i0):
        for par in range(2):
            i = i0 + par
            wait_in(par)

            @pl.when(i >= 2)
            def _():
                wait_out(par)

            transpose_block(s_v[par], d_v[par], VB // LANES)
            start_out(i, par)

            @pl.when(i < MAIN_ITERS - 2)
            def _():
                start_in(i + 2, par)

    wait_out(0)
    wait_out(1)

    @pl.when(wid < 4)
    def _():
        blk = N_MAIN_BLOCKS + wid
        pltpu.sync_copy(tT_hbm.at[:, pl.ds(blk * VB, VB)], s_v[0])
        transpose_block(s_v[0], d_v[0], VB // LANES)
        pltpu.sync_copy(d_v[0], out_hbm.at[pl.ds(blk * BLK_FLAT,
                                                 BLK_FLAT)])

    @pl.when(wid == 4)
    def _():
        pltpu.sync_copy(tT_hbm.at[:, pl.ds(N_FULL_BLOCKS * VB,
                                           V_DIM - N_FULL_BLOCKS * VB)],
                        st_v)
        transpose_block(st_v, dt_v, (V_DIM - N_FULL_BLOCKS * VB) // LANES)
        pltpu.sync_copy(
            dt_v,
            out_hbm.at[pl.ds(N_FULL_BLOCKS * BLK_FLAT,
                             (V_DIM - N_FULL_BLOCKS * VB) // 2 * PAIR_W)])


def _cbow_body(pair_hbm, off_hbm, table_hbm, out_hbm,
               pair_v, off_v, bufs, out_v, sem0, sem1, sem2, sem3):
    sems = (sem0, sem1, sem2, sem3)
    wid = lax.axis_index("s") * NC + lax.axis_index("c")

    pltpu.sync_copy(pair_hbm.at[wid], pair_v)
    pltpu.sync_copy(off_hbm.at[wid], off_v)

    def start_gather(c, b):
        pltpu.async_copy(table_hbm.at[pair_v.at[c]], bufs.at[b], sems[b])

    def wait_gather(b):
        pltpu.make_async_copy(
            table_hbm.at[pair_v.at[0]], bufs.at[b], sems[b]).wait()

    iota = lax.iota(jnp.int32, LANES)
    col_base = [iota + (g * LANES) for g in range(COL_GROUPS)]

    def reduce_chunk(c, b):
        buf = bufs.at[b]
        off_row = off_v.at[c]
        for j in range(OUT_PER_CHUNK):
            out_base = (c * OUT_PER_CHUNK + j) * EMB_DIM
            acc = [None] * COL_GROUPS
            for k in range(CTX):
                r = j * CTX + k
                rsplat = jnp.full((LANES,), r, jnp.int32)
                half = plsc.load_gather(off_row, [jnp.full((LANES,), r + 1,
                                                          jnp.int32)])
                for g in range(COL_GROUPS):
                    v = plsc.load_gather(buf, [rsplat, half + col_base[g]])
                    acc[g] = v if k == 0 else acc[g] + v
            for g in range(COL_GROUPS):
                out_v[pl.ds(out_base + g * LANES, LANES)] = acc[g] * INV_CTX

    for b in range(NBUF):
        start_gather(b, b)

    @pl.loop(0, N_CHUNKS, step=NBUF)
    def _(cc):
        for b in range(NBUF):
            c = cc + b
            wait_gather(b)
            reduce_chunk(c, b)

            @pl.when(c < N_CHUNKS - NBUF)
            def _():
                start_gather(c + NBUF, b)

    pltpu.sync_copy(out_v, out_hbm.at[pl.ds(wid * B_PER_W * EMB_DIM,
                                            B_PER_W * EMB_DIM)])


@jax.jit
def _cbow_sc(x, emb_table):
    xi = x.astype(jnp.int32)
    pair_grouped = (xi >> 1).reshape(NW, N_CHUNKS, ROWS_PER_CHUNK)
    off_grouped = jnp.pad(
        ((xi & 1) * EMB_DIM).reshape(NW, N_CHUNKS, ROWS_PER_CHUNK),
        ((0, 0), (0, 0), (1, OFF_W - ROWS_PER_CHUNK - 1)))

    mesh = plsc.VectorSubcoreMesh(core_axis_name="c", subcore_axis_name="s")

    transpose_run = pl.kernel(
        _transpose_body,
        out_type=jax.ShapeDtypeStruct((N_PAIRS * PAIR_W,), jnp.float32),
        mesh=mesh,
        scratch_types=[
            pltpu.VMEM((EMB_DIM, SRC_STRIDE), jnp.float32),
            pltpu.VMEM((EMB_DIM, SRC_STRIDE), jnp.float32),
            pltpu.VMEM((BLK_FLAT,), jnp.float32),
            pltpu.VMEM((BLK_FLAT,), jnp.float32),
            pltpu.VMEM((EMB_DIM, V_DIM - N_FULL_BLOCKS * VB), jnp.float32),
            pltpu.VMEM(((V_DIM - N_FULL_BLOCKS * VB) // 2 * PAIR_W,),
                       jnp.float32),
            pltpu.SemaphoreType.DMA,
            pltpu.SemaphoreType.DMA,
            pltpu.SemaphoreType.DMA,
            pltpu.SemaphoreType.DMA,
        ],
        compiler_params=pltpu.CompilerParams(needs_layout_passes=False),
    )
    table_pairs = transpose_run(emb_table.T).reshape(N_PAIRS, PAIR_W)

    gather_run = pl.kernel(
        _cbow_body,
        out_type=jax.ShapeDtypeStruct((BATCH * EMB_DIM,), jnp.float32),
        mesh=mesh,
        scratch_types=[
            pltpu.VMEM((N_CHUNKS, ROWS_PER_CHUNK), jnp.int32),
            pltpu.VMEM((N_CHUNKS, OFF_W), jnp.int32),
            pltpu.VMEM((NBUF, ROWS_PER_CHUNK, PAIR_W), jnp.float32),
            pltpu.VMEM((B_PER_W * EMB_DIM,), jnp.float32),
            pltpu.SemaphoreType.DMA,
            pltpu.SemaphoreType.DMA,
            pltpu.SemaphoreType.DMA,
            pltpu.SemaphoreType.DMA,
        ],
        compiler_params=pltpu.CompilerParams(needs_layout_passes=False),
    )
    out = gather_run(pair_grouped, off_grouped, table_pairs)
    return out.reshape(BATCH, EMB_DIM)


def kernel(x, emb_table):
    return _cbow_sc(x, emb_table)


# speedup vs baseline: 3.5473x; 1.0148x over previous
import functools

import jax
import jax.numpy as jnp
from jax import lax
from jax.experimental import pallas as pl
from jax.experimental.pallas import tpu as pltpu
from jax.experimental.pallas import tpu_sc as plsc

V_DIM = 1000000
EMB_DIM = 64
BATCH = 16384
CTX = 20

NC = 2
NS = 16
NW = NC * NS

LANES = 16
PAIR_W = 2 * EMB_DIM
N_PAIRS = V_DIM // 2

VB = 128
SRC_STRIDE = VB
N_FULL_BLOCKS = V_DIM // VB
MAIN_ITERS = (N_FULL_BLOCKS // NW // 2) * 2
N_MAIN_BLOCKS = MAIN_ITERS * NW
BLK_FLAT = (VB // 2) * PAIR_W

B_PER_W = BATCH // NW
OUT_PER_CHUNK = 4
ROWS_PER_CHUNK = OUT_PER_CHUNK * CTX
N_CHUNKS = B_PER_W // OUT_PER_CHUNK
NBUF = 4
COL_GROUPS = EMB_DIM // LANES
INV_CTX = 1.0 / CTX
OFF_W = 88


def _transpose_body(tT_hbm, out_hbm, s0_v, s1_v, d0_v, d1_v, st_v, dt_v,
                    sem_i0, sem_i1, sem_o0, sem_o1):
    s_v = (s0_v, s1_v)
    d_v = (d0_v, d1_v)
    sem_i = (sem_i0, sem_i1)
    sem_o = (sem_o0, sem_o1)
    wid = lax.axis_index("s") * NC + lax.axis_index("c")

    iota = lax.iota(jnp.int32, LANES)

    def start_in(i, par):
        blk = wid + NW * i
        pltpu.async_copy(tT_hbm.at[:, pl.ds(blk * VB, VB)], s_v[par],
                         sem_i[par])

    def wait_in(par):
        pltpu.make_async_copy(tT_hbm.at[:, pl.ds(0, VB)], s_v[par],
                              sem_i[par]).wait()

    def start_out(i, par):
        blk = wid + NW * i
        pltpu.async_copy(d_v[par], out_hbm.at[pl.ds(blk * BLK_FLAT,
                                                    BLK_FLAT)],
                         sem_o[par])

    def wait_out(par):
        pltpu.make_async_copy(d_v[par], out_hbm.at[pl.ds(0, BLK_FLAT)],
                              sem_o[par]).wait()

    dn = lax.GatherDimensionNumbers(offset_dims=(), collapsed_slice_dims=(0,),
                                    start_index_map=(0,))

    def lane_xor(v, s):
        return lax.gather(v, (iota ^ s).reshape(LANES, 1), dn, (1,),
                          mode=lax.GatherScatterMode.PROMISE_IN_BOUNDS)

    def transpose16(a):
        cur = a
        for s in (1, 2, 4, 8):
            nxt = [None] * LANES
            for i in range(LANES):
                keep = (iota & s) == (i & s)
                nxt[i] = jnp.where(keep, cur[i], lane_xor(cur[i ^ s], s))
            cur = nxt
        return cur

    def transpose_block(src, dst, n_vcol):
        @pl.loop(0, n_vcol)
        def _(vc):
            for frow in range(EMB_DIM // LANES):
                a = [src[LANES * frow + r, pl.ds(vc * LANES, LANES)]
                     for r in range(LANES)]
                b = transpose16(a)
                for c in range(LANES):
                    dst[pl.ds((vc * LANES + c) * EMB_DIM + LANES * frow,
                              LANES)] = b[c]

    start_in(0, 0)
    start_in(1, 1)

    @pl.loop(0, MAIN_ITERS, step=2)
    def _(i0):
        for par in range(2):
            i = i0 + par
            wait_in(par)

            @pl.when(i >= 2)
            def _():
                wait_out(par)

            transpose_block(s_v[par], d_v[par], VB // LANES)
            start_out(i, par)

            @pl.when(i < MAIN_ITERS - 2)
            def _():
                start_in(i + 2, par)

    wait_out(0)
    wait_out(1)

    @pl.when(wid < 4)
    def _():
        blk = N_MAIN_BLOCKS + wid
        pltpu.sync_copy(tT_hbm.at[:, pl.ds(blk * VB, VB)], s_v[0])
        transpose_block(s_v[0], d_v[0], VB // LANES)
        pltpu.sync_copy(d_v[0], out_hbm.at[pl.ds(blk * BLK_FLAT,
                                                 BLK_FLAT)])

    @pl.when(wid == 4)
    def _():
        pltpu.sync_copy(tT_hbm.at[:, pl.ds(N_FULL_BLOCKS * VB,
                                           V_DIM - N_FULL_BLOCKS * VB)],
                        st_v)
        transpose_block(st_v, dt_v, (V_DIM - N_FULL_BLOCKS * VB) // LANES)
        pltpu.sync_copy(
            dt_v,
            out_hbm.at[pl.ds(N_FULL_BLOCKS * BLK_FLAT,
                             (V_DIM - N_FULL_BLOCKS * VB) // 2 * PAIR_W)])


def _cbow_body(idx_hbm, table_hbm, out_hbm,
               idx_v, bufs, out_v, sem0, sem1, sem2, sem3):
    sems = (sem0, sem1, sem2, sem3)
    wid = lax.axis_index("s") * NC + lax.axis_index("c")

    pltpu.sync_copy(idx_hbm.at[wid], idx_v)

    def start_gather(c, b):
        pltpu.async_copy(table_hbm.at[idx_v.at[c]], bufs.at[b], sems[b])

    def wait_gather(b):
        pltpu.make_async_copy(
            table_hbm.at[idx_v.at[0]], bufs.at[b], sems[b]).wait()

    def reduce_chunk(c, b):
        buf = bufs.at[b]
        for j in range(OUT_PER_CHUNK):
            out_base = (c * OUT_PER_CHUNK + j) * EMB_DIM
            for g in range(COL_GROUPS):
                acc = buf[j * CTX, pl.ds(g * LANES, LANES)]
                for k in range(1, CTX):
                    acc = acc + buf[j * CTX + k, pl.ds(g * LANES, LANES)]
                out_v[pl.ds(out_base + g * LANES, LANES)] = acc * INV_CTX

    for b in range(NBUF):
        start_gather(b, b)

    @pl.loop(0, N_CHUNKS, step=NBUF)
    def _(cc):
        for b in range(NBUF):
            c = cc + b
            wait_gather(b)
            reduce_chunk(c, b)

            @pl.when(c < N_CHUNKS - NBUF)
            def _():
                start_gather(c + NBUF, b)

    pltpu.sync_copy(out_v, out_hbm.at[pl.ds(wid * B_PER_W * EMB_DIM,
                                            B_PER_W * EMB_DIM)])


@jax.jit
def _cbow_sc(x, emb_table):
    idx_grouped = x.astype(jnp.int32).reshape(NW, N_CHUNKS, ROWS_PER_CHUNK)

    mesh = plsc.VectorSubcoreMesh(core_axis_name="c", subcore_axis_name="s")

    transpose_run = pl.kernel(
        _transpose_body,
        out_type=jax.ShapeDtypeStruct((N_PAIRS * PAIR_W,), jnp.float32),
        mesh=mesh,
        scratch_types=[
            pltpu.VMEM((EMB_DIM, SRC_STRIDE), jnp.float32),
            pltpu.VMEM((EMB_DIM, SRC_STRIDE), jnp.float32),
            pltpu.VMEM((BLK_FLAT,), jnp.float32),
            pltpu.VMEM((BLK_FLAT,), jnp.float32),
            pltpu.VMEM((EMB_DIM, V_DIM - N_FULL_BLOCKS * VB), jnp.float32),
            pltpu.VMEM(((V_DIM - N_FULL_BLOCKS * VB) // 2 * PAIR_W,),
                       jnp.float32),
            pltpu.SemaphoreType.DMA,
            pltpu.SemaphoreType.DMA,
            pltpu.SemaphoreType.DMA,
            pltpu.SemaphoreType.DMA,
        ],
        compiler_params=pltpu.CompilerParams(needs_layout_passes=False),
    )
    table_rows = transpose_run(emb_table.T).reshape(V_DIM, EMB_DIM)

    gather_run = pl.kernel(
        _cbow_body,
        out_type=jax.ShapeDtypeStruct((BATCH * EMB_DIM,), jnp.float32),
        mesh=mesh,
        scratch_types=[
            pltpu.VMEM((N_CHUNKS, ROWS_PER_CHUNK), jnp.int32),
            pltpu.VMEM((NBUF, ROWS_PER_CHUNK, EMB_DIM), jnp.float32),
            pltpu.VMEM((B_PER_W * EMB_DIM,), jnp.float32),
            pltpu.SemaphoreType.DMA,
            pltpu.SemaphoreType.DMA,
            pltpu.SemaphoreType.DMA,
            pltpu.SemaphoreType.DMA,
        ],
        compiler_params=pltpu.CompilerParams(use_tc_tiling_on_sc=False),
    )
    out = gather_run(idx_grouped, table_rows)
    return out.reshape(BATCH, EMB_DIM)


def kernel(x, emb_table):
    return _cbow_sc(x, emb_table)


# speedup vs baseline: 3.5589x; 1.0033x over previous
import jax
import jax.numpy as jnp
from jax import lax
from jax.experimental import pallas as pl
from jax.experimental.pallas import tpu as pltpu
from jax.experimental.pallas import tpu_sc as plsc

V_DIM = 1000000
EMB_DIM = 64
BATCH = 16384
CTX = 20

NC = 2
NS = 16
NW = NC * NS

LANES = 16

VB = 128
N_FULL_BLOCKS = V_DIM // VB
MAIN_ITERS = (N_FULL_BLOCKS // NW // 2) * 2
N_MAIN_BLOCKS = MAIN_ITERS * NW
BLK_FLAT = VB * EMB_DIM

B_PER_W = BATCH // NW
OUT_PER_CHUNK = 4
ROWS_PER_CHUNK = OUT_PER_CHUNK * CTX
N_CHUNKS = B_PER_W // OUT_PER_CHUNK
NBUF = 4
COL_GROUPS = EMB_DIM // LANES
INV_CTX = 1.0 / CTX


def _transpose_body(tT_hbm, out_hbm, s0_v, s1_v, d0_v, d1_v, st_v, dt_v,
                    sem_i0, sem_i1, sem_o0, sem_o1):
    s_v = (s0_v, s1_v)
    d_v = (d0_v, d1_v)
    sem_i = (sem_i0, sem_i1)
    sem_o = (sem_o0, sem_o1)
    wid = lax.axis_index("s") * NC + lax.axis_index("c")

    iota = lax.iota(jnp.int32, LANES)

    def start_in(i, par):
        blk = wid + NW * i
        pltpu.async_copy(tT_hbm.at[:, pl.ds(blk * VB, VB)], s_v[par],
                         sem_i[par])

    def wait_in(par):
        pltpu.make_async_copy(tT_hbm.at[:, pl.ds(0, VB)], s_v[par],
                              sem_i[par]).wait()

    def start_out(i, par):
        blk = wid + NW * i
        pltpu.async_copy(d_v[par], out_hbm.at[pl.ds(blk * BLK_FLAT,
                                                    BLK_FLAT)],
                         sem_o[par])

    def wait_out(par):
        pltpu.make_async_copy(d_v[par], out_hbm.at[pl.ds(0, BLK_FLAT)],
                              sem_o[par]).wait()

    dn = lax.GatherDimensionNumbers(offset_dims=(), collapsed_slice_dims=(0,),
                                    start_index_map=(0,))

    def lane_xor(v, s):
        return lax.gather(v, (iota ^ s).reshape(LANES, 1), dn, (1,),
                          mode=lax.GatherScatterMode.PROMISE_IN_BOUNDS)

    def transpose16(a):
        cur = a
        for s in (1, 2, 4, 8):
            nxt = [None] * LANES
            for i in range(LANES):
                keep = (iota & s) == (i & s)
                nxt[i] = jnp.where(keep, cur[i], lane_xor(cur[i ^ s], s))
            cur = nxt
        return cur

    def transpose_block(src, dst, n_vcol):
        @pl.loop(0, n_vcol)
        def _(vc):
            for frow in range(EMB_DIM // LANES):
                a = [src[LANES * frow + r, pl.ds(vc * LANES, LANES)]
                     for r in range(LANES)]
                b = transpose16(a)
                for c in range(LANES):
                    dst[pl.ds((vc * LANES + c) * EMB_DIM + LANES * frow,
                              LANES)] = b[c]

    start_in(0, 0)
    start_in(1, 1)

    @pl.loop(0, MAIN_ITERS, step=2)
    def _(i0):
        for par in range(2):
            i = i0 + par
            wait_in(par)

            @pl.when(i >= 2)
            def _():
                wait_out(par)

            transpose_block(s_v[par], d_v[par], VB // LANES)
            start_out(i, par)

            @pl.when(i < MAIN_ITERS - 2)
            def _():
                start_in(i + 2, par)

    wait_out(0)
    wait_out(1)

    @pl.when(wid < 4)
    def _():
        blk = N_MAIN_BLOCKS + wid
        pltpu.sync_copy(tT_hbm.at[:, pl.ds(blk * VB, VB)], s_v[0])
        transpose_block(s_v[0], d_v[0], VB // LANES)
        pltpu.sync_copy(d_v[0], out_hbm.at[pl.ds(blk * BLK_FLAT,
                                                 BLK_FLAT)])

    @pl.when(wid == 4)
    def _():
        pltpu.sync_copy(tT_hbm.at[:, pl.ds(N_FULL_BLOCKS * VB,
                                           V_DIM - N_FULL_BLOCKS * VB)],
                        st_v)
        transpose_block(st_v, dt_v, (V_DIM - N_FULL_BLOCKS * VB) // LANES)
        pltpu.sync_copy(
            dt_v,
            out_hbm.at[pl.ds(N_FULL_BLOCKS * BLK_FLAT,
                             (V_DIM - N_FULL_BLOCKS * VB) * EMB_DIM)])


def _cbow_body(idx_hbm, table_hbm, out_hbm,
               idx_v, bufs, out_v, sem0, sem1, sem2, sem3):
    sems = (sem0, sem1, sem2, sem3)
    wid = lax.axis_index("s") * NC + lax.axis_index("c")

    pltpu.sync_copy(idx_hbm.at[wid], idx_v)

    def start_gather(c, b):
        pltpu.async_copy(table_hbm.at[idx_v.at[c]], bufs.at[b], sems[b])

    def wait_gather(b):
        pltpu.make_async_copy(
            table_hbm.at[idx_v.at[0]], bufs.at[b], sems[b]).wait()

    def reduce_chunk(c, b):
        buf = bufs.at[b]
        for j in range(OUT_PER_CHUNK):
            out_base = (c * OUT_PER_CHUNK + j) * EMB_DIM
            for g in range(COL_GROUPS):
                acc = buf[j * CTX, pl.ds(g * LANES, LANES)]
                for k in range(1, CTX):
                    acc = acc + buf[j * CTX + k, pl.ds(g * LANES, LANES)]
                out_v[pl.ds(out_base + g * LANES, LANES)] = acc * INV_CTX

    for b in range(NBUF):
        start_gather(b, b)

    @pl.loop(0, N_CHUNKS, step=NBUF)
    def _(cc):
        for b in range(NBUF):
            c = cc + b
            wait_gather(b)
            reduce_chunk(c, b)

            @pl.when(c < N_CHUNKS - NBUF)
            def _():
                start_gather(c + NBUF, b)

    pltpu.sync_copy(out_v, out_hbm.at[pl.ds(wid * B_PER_W * EMB_DIM,
                                            B_PER_W * EMB_DIM)])


@jax.jit
def _cbow_sc(x, emb_table):
    idx_grouped = x.astype(jnp.int32).reshape(NW, N_CHUNKS, ROWS_PER_CHUNK)

    mesh = plsc.VectorSubcoreMesh(core_axis_name="c", subcore_axis_name="s")

    transpose_run = pl.kernel(
        _transpose_body,
        out_type=jax.ShapeDtypeStruct((V_DIM * EMB_DIM,), jnp.float32),
        mesh=mesh,
        scratch_types=[
            pltpu.VMEM((EMB_DIM, VB), jnp.float32),
            pltpu.VMEM((EMB_DIM, VB), jnp.float32),
            pltpu.VMEM((BLK_FLAT,), jnp.float32),
            pltpu.VMEM((BLK_FLAT,), jnp.float32),
            pltpu.VMEM((EMB_DIM, V_DIM - N_FULL_BLOCKS * VB), jnp.float32),
            pltpu.VMEM(((V_DIM - N_FULL_BLOCKS * VB) * EMB_DIM,),
                       jnp.float32),
            pltpu.SemaphoreType.DMA,
            pltpu.SemaphoreType.DMA,
            pltpu.SemaphoreType.DMA,
            pltpu.SemaphoreType.DMA,
        ],
        compiler_params=pltpu.CompilerParams(needs_layout_passes=False),
    )
    table_rows = transpose_run(emb_table.T).reshape(V_DIM, EMB_DIM)

    gather_run = pl.kernel(
        _cbow_body,
        out_type=jax.ShapeDtypeStruct((BATCH * EMB_DIM,), jnp.float32),
        mesh=mesh,
        scratch_types=[
            pltpu.VMEM((N_CHUNKS, ROWS_PER_CHUNK), jnp.int32),
            pltpu.VMEM((NBUF, ROWS_PER_CHUNK, EMB_DIM), jnp.float32),
            pltpu.VMEM((B_PER_W * EMB_DIM,), jnp.float32),
            pltpu.SemaphoreType.DMA,
            pltpu.SemaphoreType.DMA,
            pltpu.SemaphoreType.DMA,
            pltpu.SemaphoreType.DMA,
        ],
        compiler_params=pltpu.CompilerParams(use_tc_tiling_on_sc=False),
    )
    out = gather_run(idx_grouped, table_rows)
    return out.reshape(BATCH, EMB_DIM)


def kernel(x, emb_table):
    return _cbow_sc(x, emb_table)
